# baseline (speedup 1.0000x reference)
import functools

import jax
import jax.numpy as jnp
from jax import lax
from jax.experimental import pallas as pl
from jax.experimental.pallas import tpu as pltpu
from jax.experimental.pallas import tpu_sc as plsc

_N = 10000
_E = 160000
_DN = 128
_DE = 16
_EO = 64
_NO = 128

_NC = 2
_NS = 16
_NW = _NC * _NS
_PER_W = _E // _NW
_C = 40
_NCH = _PER_W // _C
_NP = 10240
_RPT = _NP // _NS

_EB = 2048
_NB = 2048

@functools.cache
def _sc_gather_kernel():
    mesh = plsc.VectorSubcoreMesh(core_axis_name="c", subcore_axis_name="s",
                                  num_cores=_NC, num_subcores=_NS)

    @functools.partial(
        pl.kernel,
        out_type=[
            jax.ShapeDtypeStruct((_E, _DN), jnp.float32),
            jax.ShapeDtypeStruct((_E, _DN), jnp.float32),
        ],
        mesh=mesh,
        scratch_types=[
            pltpu.VMEM((_NCH, _C), jnp.int32),
            pltpu.VMEM((_NCH, _C), jnp.int32),
            pltpu.VMEM((_C, _DN), jnp.float32),
            pltpu.VMEM((_C, _DN), jnp.float32),
            pltpu.VMEM((_C, _DN), jnp.float32),
            pltpu.VMEM((_C, _DN), jnp.float32),
            pltpu.SemaphoreType.DMA,
            pltpu.SemaphoreType.DMA,
            pltpu.SemaphoreType.DMA,
            pltpu.SemaphoreType.DMA,
        ],
    )
    def _sc_gather(h_hbm, ridx_hbm, cidx_hbm, outr_hbm, outc_hbm,
                   ridx_v, cidx_v, bufr0, bufc0, bufr1, bufc1,
                   sg0, sg1, sw0, sw1):
        wid = lax.axis_index("c") * _NS + lax.axis_index("s")
        base = wid * _PER_W
        bufrs = (bufr0, bufr1)
        bufcs = (bufc0, bufc1)
        sgs = (sg0, sg1)
        sws = (sw0, sw1)
        pltpu.sync_copy(ridx_hbm.at[wid], ridx_v)
        pltpu.sync_copy(cidx_hbm.at[wid], cidx_v)

        def issue_g(j, b):
            pltpu.async_copy(h_hbm.at[ridx_v.at[j]], bufrs[b], sgs[b])
            pltpu.async_copy(h_hbm.at[cidx_v.at[j]], bufcs[b], sgs[b])

        def drain_g(b):
            pltpu.make_async_copy(h_hbm.at[pl.ds(0, _C)], bufrs[b],
                                  sgs[b]).wait()
            pltpu.make_async_copy(h_hbm.at[pl.ds(0, _C)], bufcs[b],
                                  sgs[b]).wait()

        def write(j, b):
            pltpu.async_copy(bufrs[b], outr_hbm.at[pl.ds(base + j * _C, _C)],
                             sws[b])
            pltpu.async_copy(bufcs[b], outc_hbm.at[pl.ds(base + j * _C, _C)],
                             sws[b])
            pltpu.make_async_copy(bufrs[b], outr_hbm.at[pl.ds(0, _C)],
                                  sws[b]).wait()
            pltpu.make_async_copy(bufcs[b], outc_hbm.at[pl.ds(0, _C)],
                                  sws[b]).wait()

        issue_g(0, 0)

        def pair(j2, carry):
            j = j2 * 2

            @pl.when(j + 1 < _NCH)
            def _():
                issue_g(j + 1, 1)

            drain_g(0)
            write(j, 0)

            @pl.when(j + 2 < _NCH)
            def _():
                issue_g(j + 2, 0)

            @pl.when(j + 1 < _NCH)
            def _():
                drain_g(1)
                write(j + 1, 1)

            return carry

        lax.fori_loop(0, (_NCH + 1) // 2, pair, 0)

    return _sc_gather


_EG = 4
_FG = 8
_FPT = _EO // _FG
_SCH = 128
_EGB = 39936
_ACC = _FPT * _NP


@functools.cache
def _sc_scatter_kernel():
    mesh = plsc.VectorSubcoreMesh(core_axis_name="c", subcore_axis_name="s",
                                  num_cores=_NC, num_subcores=_NS)

    @functools.partial(
        pl.kernel,
        out_type=jax.ShapeDtypeStruct((_NW, _ACC), jnp.float32),
        mesh=mesh,
        compiler_params=pltpu.CompilerParams(needs_layout_passes=False),
        scratch_types=[
            pltpu.VMEM((_ACC,), jnp.float32),
            pltpu.VMEM((_SCH,), jnp.int32),
            pltpu.VMEM((_SCH,), jnp.int32),
            pltpu.VMEM((_FPT, _SCH), jnp.float32),
            pltpu.VMEM((_FPT, _SCH), jnp.float32),
            pltpu.SemaphoreType.DMA,
            pltpu.SemaphoreType.DMA,
            pltpu.SemaphoreType.DMA,
            pltpu.SemaphoreType.DMA,
        ],
    )
    def _sc_scatter(et_hbm, cidx_hbm, out_hbm, acc, idxb0, idxb1,
                    ebuf0, ebuf1, si0, si1, se0, se1):
        t = lax.axis_index("c") * _NS + lax.axis_index("s")
        fg = t // _EG
        eg = t % _EG
        base = eg * _EGB
        f0 = fg * _FPT
        nch = jnp.where(eg == _EG - 1, 314, 312)
        sis = (si0, si1)
        ses = (se0, se1)
        idxbs = (idxb0, idxb1)
        ebufs = (ebuf0, ebuf1)

        def zero(v, carry):
            acc[pl.ds(v * 16, 16)] = jnp.zeros((16,), jnp.float32)
            return carry

        lax.fori_loop(0, _ACC // 16, zero, 0)

        def issue(g, b):
            off = base + g * _SCH
            pltpu.async_copy(cidx_hbm.at[pl.ds(off, _SCH)], idxbs[b], sis[b])
            pltpu.async_copy(et_hbm.at[pl.ds(f0, _FPT), pl.ds(off, _SCH)],
                             ebufs[b], ses[b])

        def drain(b):
            pltpu.make_async_copy(cidx_hbm.at[pl.ds(0, _SCH)], idxbs[b],
                                  sis[b]).wait()
            pltpu.make_async_copy(et_hbm.at[pl.ds(0, _FPT), pl.ds(0, _SCH)],
                                  ebufs[b], ses[b]).wait()

        def compute(b):
            def inner(v, c2):
                r_vec = idxbs[b][pl.ds(v * 16, 16)]
                for k in range(_FPT):
                    vals = ebufs[b][k, pl.ds(v * 16, 16)]
                    plsc.addupdate_scatter(acc, [r_vec + k * _NP], vals)
                return c2

            lax.fori_loop(0, _SCH // 16, inner, 0)

        issue(0, 0)

        def pair(g2, carry):
            g = g2 * 2

            @pl.when(g + 1 < nch)
            def _():
                issue(g + 1, 1)

            drain(0)
            compute(0)

            @pl.when(g + 2 < nch)
            def _():
                issue(g + 2, 0)

            @pl.when(g + 1 < nch)
            def _():
                drain(1)
                compute(1)

            return carry

        lax.fori_loop(0, (nch + 1) // 2, pair, 0)
        pltpu.sync_copy(acc, out_hbm.at[t])

    return _sc_scatter


def _edge_body(hr, hc, ea, w1r, w1c, w1e, b1, w2, b2, w3, b3, w4, b4, out, outt):
    f32 = jnp.float32
    bf = jnp.bfloat16
    x = (jnp.dot(hr[...].astype(bf), w1r[...], preferred_element_type=f32)
         + jnp.dot(hc[...].astype(bf), w1c[...], preferred_element_type=f32)
         + jnp.dot(ea[...].astype(bf), w1e[...], preferred_element_type=f32)
         + b1[...])
    x = jnp.maximum(x, 0.0).astype(bf)
    x = jnp.maximum(jnp.dot(x, w2[...], preferred_element_type=f32) + b2[...],
                    0.0).astype(bf)
    x = jnp.maximum(jnp.dot(x, w3[...], preferred_element_type=f32) + b3[...],
                    0.0).astype(bf)
    e = jnp.dot(x, w4[...], preferred_element_type=f32) + b4[...]
    out[...] = e
    outt[...] = e.T


def _full(shape):
    return pl.BlockSpec(shape, lambda i: (0, 0))


def _edge_mlp(hr_g, hc_g, ea, w1r, w1c, w1e, b1, w2, b2, w3, b3, w4, b4):
    grid = (_E + _EB - 1) // _EB
    return pl.pallas_call(
        _edge_body,
        grid=(grid,),
        in_specs=[
            pl.BlockSpec((_EB, _DN), lambda i: (i, 0)),
            pl.BlockSpec((_EB, _DN), lambda i: (i, 0)),
            pl.BlockSpec((_EB, _DE), lambda i: (i, 0)),
            _full(w1r.shape), _full(w1c.shape), _full(w1e.shape), _full(b1.shape),
            _full(w2.shape), _full(b2.shape),
            _full(w3.shape), _full(b3.shape),
            _full(w4.shape), _full(b4.shape),
        ],
        out_specs=[
            pl.BlockSpec((_EB, _EO), lambda i: (i, 0)),
            pl.BlockSpec((_EO, _EB), lambda i: (0, i)),
        ],
        out_shape=[
            jax.ShapeDtypeStruct((_E, _EO), jnp.float32),
            jax.ShapeDtypeStruct((_EO, _E), jnp.float32),
        ],
    )(hr_g, hc_g, ea, w1r, w1c, w1e, b1, w2, b2, w3, b3, w4, b4)


def _node_body(parts, h, w1m, w1h, b1, w2, b2, w3, b3, w4, b4, out):
    f32 = jnp.float32
    p = parts[...].reshape(_FG, _EG, _FPT, _NB)
    msum = jnp.sum(p, axis=1).reshape(_EO, _NB)
    x = (lax.dot_general(msum, w1m[...], (((0,), (0,)), ((), ())),
                         preferred_element_type=f32)
         + jnp.dot(h[...], w1h[...], preferred_element_type=f32)
         + b1[...])
    x = jnp.maximum(x, 0.0)
    x = jnp.maximum(jnp.dot(x, w2[...], preferred_element_type=f32) + b2[...], 0.0)
    x = jnp.maximum(jnp.dot(x, w3[...], preferred_element_type=f32) + b3[...], 0.0)
    out[...] = jnp.dot(x, w4[...], preferred_element_type=f32) + b4[...]


def _node_mlp(parts, h, w1m, w1h, b1, w2, b2, w3, b3, w4, b4):
    grid = _NP // _NB
    return pl.pallas_call(
        _node_body,
        grid=(grid,),
        in_specs=[
            pl.BlockSpec((_NW * _FPT, _NB), lambda i: (0, i)),
            pl.BlockSpec((_NB, _DN), lambda i: (i, 0)),
            _full(w1m.shape), _full(w1h.shape), _full(b1.shape),
            _full(w2.shape), _full(b2.shape),
            _full(w3.shape), _full(b3.shape),
            _full(w4.shape), _full(b4.shape),
        ],
        out_specs=pl.BlockSpec((_NB, _NO), lambda i: (i, 0)),
        out_shape=jax.ShapeDtypeStruct((_N, _NO), jnp.float32),
    )(parts, h, w1m, w1h, b1, w2, b2, w3, b3, w4, b4)


def kernel(h, edge_index, edge_attr, edge_params, node_params):
    row3 = edge_index[0].reshape(_NW, _NCH, _C)
    col3 = edge_index[1].reshape(_NW, _NCH, _C)

    hr_g, hc_g = _sc_gather_kernel()(h, row3, col3)

    (ew1, eb1), (ew2, eb2), (ew3, eb3), (ew4, eb4) = edge_params
    bf = jnp.bfloat16
    e_new, e_newt = _edge_mlp(
        hr_g, hc_g, edge_attr,
        ew1[:_DN].astype(bf), ew1[_DN:2 * _DN].astype(bf),
        ew1[2 * _DN:].astype(bf), eb1.reshape(1, -1),
        ew2.astype(bf), eb2.reshape(1, -1), ew3.astype(bf), eb3.reshape(1, -1),
        ew4.astype(bf), eb4.reshape(1, -1),
    )

    parts = _sc_scatter_kernel()(e_newt, edge_index[1])
    parts = parts.reshape(_NW * _FPT, _NP)

    (nw1, nb1), (nw2, nb2), (nw3, nb3), (nw4, nb4) = node_params
    h_new = _node_mlp(
        parts, h,
        nw1[:_EO], nw1[_EO:], nb1.reshape(1, -1),
        nw2, nb2.reshape(1, -1), nw3, nb3.reshape(1, -1), nw4, nb4.reshape(1, -1),
    )
    return (h_new, e_new)


# speedup vs baseline: 1.0242x; 1.0242x over previous
import functools

import jax
import jax.numpy as jnp
from jax import lax
from jax.experimental import pallas as pl
from jax.experimental.pallas import tpu as pltpu
from jax.experimental.pallas import tpu_sc as plsc

_N = 10000
_E = 160000
_DN = 128
_DE = 16
_EO = 64
_NO = 128

_NC = 2
_NS = 16
_NW = _NC * _NS
_PER_W = _E // _NW
_C = 40
_NCH = _PER_W // _C
_NP = 10240
_RPT = _NP // _NS

_EB = 4096
_NB = 2048

@functools.cache
def _sc_gather_kernel():
    mesh = plsc.VectorSubcoreMesh(core_axis_name="c", subcore_axis_name="s",
                                  num_cores=_NC, num_subcores=_NS)

    @functools.partial(
        pl.kernel,
        out_type=[
            jax.ShapeDtypeStruct((_E, _DN), jnp.float32),
            jax.ShapeDtypeStruct((_E, _DN), jnp.float32),
        ],
        mesh=mesh,
        scratch_types=[
            pltpu.VMEM((_NCH, _C), jnp.int32),
            pltpu.VMEM((_NCH, _C), jnp.int32),
            pltpu.VMEM((_C, _DN), jnp.float32),
            pltpu.VMEM((_C, _DN), jnp.float32),
            pltpu.VMEM((_C, _DN), jnp.float32),
            pltpu.VMEM((_C, _DN), jnp.float32),
            pltpu.SemaphoreType.DMA,
            pltpu.SemaphoreType.DMA,
            pltpu.SemaphoreType.DMA,
            pltpu.SemaphoreType.DMA,
        ],
    )
    def _sc_gather(h_hbm, ridx_hbm, cidx_hbm, outr_hbm, outc_hbm,
                   ridx_v, cidx_v, bufr0, bufc0, bufr1, bufc1,
                   sg0, sg1, sw0, sw1):
        wid = lax.axis_index("c") * _NS + lax.axis_index("s")
        base = wid * _PER_W
        bufrs = (bufr0, bufr1)
        bufcs = (bufc0, bufc1)
        sgs = (sg0, sg1)
        sws = (sw0, sw1)
        pltpu.sync_copy(ridx_hbm.at[wid], ridx_v)
        pltpu.sync_copy(cidx_hbm.at[wid], cidx_v)

        def issue_g(j, b):
            pltpu.async_copy(h_hbm.at[ridx_v.at[j]], bufrs[b], sgs[b])
            pltpu.async_copy(h_hbm.at[cidx_v.at[j]], bufcs[b], sgs[b])

        def drain_g(b):
            pltpu.make_async_copy(h_hbm.at[pl.ds(0, _C)], bufrs[b],
                                  sgs[b]).wait()
            pltpu.make_async_copy(h_hbm.at[pl.ds(0, _C)], bufcs[b],
                                  sgs[b]).wait()

        def write(j, b):
            pltpu.async_copy(bufrs[b], outr_hbm.at[pl.ds(base + j * _C, _C)],
                             sws[b])
            pltpu.async_copy(bufcs[b], outc_hbm.at[pl.ds(base + j * _C, _C)],
                             sws[b])
            pltpu.make_async_copy(bufrs[b], outr_hbm.at[pl.ds(0, _C)],
                                  sws[b]).wait()
            pltpu.make_async_copy(bufcs[b], outc_hbm.at[pl.ds(0, _C)],
                                  sws[b]).wait()

        issue_g(0, 0)

        def pair(j2, carry):
            j = j2 * 2

            @pl.when(j + 1 < _NCH)
            def _():
                issue_g(j + 1, 1)

            drain_g(0)
            write(j, 0)

            @pl.when(j + 2 < _NCH)
            def _():
                issue_g(j + 2, 0)

            @pl.when(j + 1 < _NCH)
            def _():
                drain_g(1)
                write(j + 1, 1)

            return carry

        lax.fori_loop(0, (_NCH + 1) // 2, pair, 0)

    return _sc_gather


_EG = 4
_FG = 8
_FPT = _EO // _FG
_SCH = 128
_EGB = 39936
_ACC = _FPT * _NP


@functools.cache
def _sc_scatter_kernel():
    mesh = plsc.VectorSubcoreMesh(core_axis_name="c", subcore_axis_name="s",
                                  num_cores=_NC, num_subcores=_NS)

    @functools.partial(
        pl.kernel,
        out_type=jax.ShapeDtypeStruct((_NW, _ACC), jnp.float32),
        mesh=mesh,
        compiler_params=pltpu.CompilerParams(needs_layout_passes=False),
        scratch_types=[
            pltpu.VMEM((_ACC,), jnp.float32),
            pltpu.VMEM((_SCH,), jnp.int32),
            pltpu.VMEM((_SCH,), jnp.int32),
            pltpu.VMEM((_FPT, _SCH), jnp.float32),
            pltpu.VMEM((_FPT, _SCH), jnp.float32),
            pltpu.SemaphoreType.DMA,
            pltpu.SemaphoreType.DMA,
            pltpu.SemaphoreType.DMA,
            pltpu.SemaphoreType.DMA,
        ],
    )
    def _sc_scatter(et_hbm, cidx_hbm, out_hbm, acc, idxb0, idxb1,
                    ebuf0, ebuf1, si0, si1, se0, se1):
        t = lax.axis_index("c") * _NS + lax.axis_index("s")
        fg = t // _EG
        eg = t % _EG
        base = eg * _EGB
        f0 = fg * _FPT
        nch = jnp.where(eg == _EG - 1, 314, 312)
        sis = (si0, si1)
        ses = (se0, se1)
        idxbs = (idxb0, idxb1)
        ebufs = (ebuf0, ebuf1)

        def zero(v, carry):
            acc[pl.ds(v * 16, 16)] = jnp.zeros((16,), jnp.float32)
            return carry

        lax.fori_loop(0, _ACC // 16, zero, 0)

        def issue(g, b):
            off = base + g * _SCH
            pltpu.async_copy(cidx_hbm.at[pl.ds(off, _SCH)], idxbs[b], sis[b])
            pltpu.async_copy(et_hbm.at[pl.ds(f0, _FPT), pl.ds(off, _SCH)],
                             ebufs[b], ses[b])

        def drain(b):
            pltpu.make_async_copy(cidx_hbm.at[pl.ds(0, _SCH)], idxbs[b],
                                  sis[b]).wait()
            pltpu.make_async_copy(et_hbm.at[pl.ds(0, _FPT), pl.ds(0, _SCH)],
                                  ebufs[b], ses[b]).wait()

        def compute(b):
            def inner(v, c2):
                r_vec = idxbs[b][pl.ds(v * 16, 16)]
                for k in range(_FPT):
                    vals = ebufs[b][k, pl.ds(v * 16, 16)]
                    plsc.addupdate_scatter(acc, [r_vec + k * _NP], vals)
                return c2

            lax.fori_loop(0, _SCH // 16, inner, 0)

        issue(0, 0)

        def pair(g2, carry):
            g = g2 * 2

            @pl.when(g + 1 < nch)
            def _():
                issue(g + 1, 1)

            drain(0)
            compute(0)

            @pl.when(g + 2 < nch)
            def _():
                issue(g + 2, 0)

            @pl.when(g + 1 < nch)
            def _():
                drain(1)
                compute(1)

            return carry

        lax.fori_loop(0, (nch + 1) // 2, pair, 0)
        pltpu.sync_copy(acc, out_hbm.at[t])

    return _sc_scatter


def _edge_body(hr, hc, ea, w1r, w1c, w1e, b1, w2, b2, w3, b3, w4, b4, out, outt):
    f32 = jnp.float32
    bf = jnp.bfloat16
    x = (jnp.dot(hr[...].astype(bf), w1r[...], preferred_element_type=f32)
         + jnp.dot(hc[...].astype(bf), w1c[...], preferred_element_type=f32)
         + jnp.dot(ea[...].astype(bf), w1e[...], preferred_element_type=f32)
         + b1[...])
    x = jnp.maximum(x, 0.0).astype(bf)
    x = jnp.maximum(jnp.dot(x, w2[...], preferred_element_type=f32) + b2[...],
                    0.0).astype(bf)
    x = jnp.maximum(jnp.dot(x, w3[...], preferred_element_type=f32) + b3[...],
                    0.0).astype(bf)
    e = jnp.dot(x, w4[...], preferred_element_type=f32) + b4[...]
    out[...] = e
    outt[...] = e.T


def _full(shape):
    return pl.BlockSpec(shape, lambda i: (0, 0))


def _edge_mlp(hr_g, hc_g, ea, w1r, w1c, w1e, b1, w2, b2, w3, b3, w4, b4):
    grid = (_E + _EB - 1) // _EB
    return pl.pallas_call(
        _edge_body,
        grid=(grid,),
        in_specs=[
            pl.BlockSpec((_EB, _DN), lambda i: (i, 0)),
            pl.BlockSpec((_EB, _DN), lambda i: (i, 0)),
            pl.BlockSpec((_EB, _DE), lambda i: (i, 0)),
            _full(w1r.shape), _full(w1c.shape), _full(w1e.shape), _full(b1.shape),
            _full(w2.shape), _full(b2.shape),
            _full(w3.shape), _full(b3.shape),
            _full(w4.shape), _full(b4.shape),
        ],
        out_specs=[
            pl.BlockSpec((_EB, _EO), lambda i: (i, 0)),
            pl.BlockSpec((_EO, _EB), lambda i: (0, i)),
        ],
        out_shape=[
            jax.ShapeDtypeStruct((_E, _EO), jnp.float32),
            jax.ShapeDtypeStruct((_EO, _E), jnp.float32),
        ],
    )(hr_g, hc_g, ea, w1r, w1c, w1e, b1, w2, b2, w3, b3, w4, b4)


def _node_body(parts, h, w1m, w1h, b1, w2, b2, w3, b3, w4, b4, out):
    f32 = jnp.float32
    p = parts[...].reshape(_FG, _EG, _FPT, _NB)
    msum = jnp.sum(p, axis=1).reshape(_EO, _NB)
    x = (lax.dot_general(msum, w1m[...], (((0,), (0,)), ((), ())),
                         preferred_element_type=f32)
         + jnp.dot(h[...], w1h[...], preferred_element_type=f32)
         + b1[...])
    x = jnp.maximum(x, 0.0)
    x = jnp.maximum(jnp.dot(x, w2[...], preferred_element_type=f32) + b2[...], 0.0)
    x = jnp.maximum(jnp.dot(x, w3[...], preferred_element_type=f32) + b3[...], 0.0)
    out[...] = jnp.dot(x, w4[...], preferred_element_type=f32) + b4[...]


def _node_mlp(parts, h, w1m, w1h, b1, w2, b2, w3, b3, w4, b4):
    grid = _NP // _NB
    return pl.pallas_call(
        _node_body,
        grid=(grid,),
        in_specs=[
            pl.BlockSpec((_NW * _FPT, _NB), lambda i: (0, i)),
            pl.BlockSpec((_NB, _DN), lambda i: (i, 0)),
            _full(w1m.shape), _full(w1h.shape), _full(b1.shape),
            _full(w2.shape), _full(b2.shape),
            _full(w3.shape), _full(b3.shape),
            _full(w4.shape), _full(b4.shape),
        ],
        out_specs=pl.BlockSpec((_NB, _NO), lambda i: (i, 0)),
        out_shape=jax.ShapeDtypeStruct((_N, _NO), jnp.float32),
    )(parts, h, w1m, w1h, b1, w2, b2, w3, b3, w4, b4)


def kernel(h, edge_index, edge_attr, edge_params, node_params):
    row3 = edge_index[0].reshape(_NW, _NCH, _C)
    col3 = edge_index[1].reshape(_NW, _NCH, _C)

    hr_g, hc_g = _sc_gather_kernel()(h, row3, col3)

    (ew1, eb1), (ew2, eb2), (ew3, eb3), (ew4, eb4) = edge_params
    bf = jnp.bfloat16
    e_new, e_newt = _edge_mlp(
        hr_g, hc_g, edge_attr,
        ew1[:_DN].astype(bf), ew1[_DN:2 * _DN].astype(bf),
        ew1[2 * _DN:].astype(bf), eb1.reshape(1, -1),
        ew2.astype(bf), eb2.reshape(1, -1), ew3.astype(bf), eb3.reshape(1, -1),
        ew4.astype(bf), eb4.reshape(1, -1),
    )

    parts = _sc_scatter_kernel()(e_newt, edge_index[1])
    parts = parts.reshape(_NW * _FPT, _NP)

    (nw1, nb1), (nw2, nb2), (nw3, nb3), (nw4, nb4) = node_params
    h_new = _node_mlp(
        parts, h,
        nw1[:_EO], nw1[_EO:], nb1.reshape(1, -1),
        nw2, nb2.reshape(1, -1), nw3, nb3.reshape(1, -1), nw4, nb4.reshape(1, -1),
    )
    return (h_new, e_new)


# speedup vs baseline: 1.0343x; 1.0099x over previous
import functools

import jax
import jax.numpy as jnp
from jax import lax
from jax.experimental import pallas as pl
from jax.experimental.pallas import tpu as pltpu
from jax.experimental.pallas import tpu_sc as plsc

_N = 10000
_E = 160000
_DN = 128
_DE = 16
_EO = 64
_NO = 128

_NC = 2
_NS = 16
_NW = _NC * _NS
_PER_W = _E // _NW
_C = 40
_NCH = _PER_W // _C
_NP = 10240
_RPT = _NP // _NS

_EB = 8192
_NB = 2048

@functools.cache
def _sc_gather_kernel():
    mesh = plsc.VectorSubcoreMesh(core_axis_name="c", subcore_axis_name="s",
                                  num_cores=_NC, num_subcores=_NS)

    @functools.partial(
        pl.kernel,
        out_type=[
            jax.ShapeDtypeStruct((_E, _DN), jnp.float32),
            jax.ShapeDtypeStruct((_E, _DN), jnp.float32),
        ],
        mesh=mesh,
        scratch_types=[
            pltpu.VMEM((_NCH, _C), jnp.int32),
            pltpu.VMEM((_NCH, _C), jnp.int32),
            pltpu.VMEM((_C, _DN), jnp.float32),
            pltpu.VMEM((_C, _DN), jnp.float32),
            pltpu.VMEM((_C, _DN), jnp.float32),
            pltpu.VMEM((_C, _DN), jnp.float32),
            pltpu.SemaphoreType.DMA,
            pltpu.SemaphoreType.DMA,
            pltpu.SemaphoreType.DMA,
            pltpu.SemaphoreType.DMA,
        ],
    )
    def _sc_gather(h_hbm, ridx_hbm, cidx_hbm, outr_hbm, outc_hbm,
                   ridx_v, cidx_v, bufr0, bufc0, bufr1, bufc1,
                   sg0, sg1, sw0, sw1):
        wid = lax.axis_index("c") * _NS + lax.axis_index("s")
        base = wid * _PER_W
        bufrs = (bufr0, bufr1)
        bufcs = (bufc0, bufc1)
        sgs = (sg0, sg1)
        sws = (sw0, sw1)
        pltpu.sync_copy(ridx_hbm.at[wid], ridx_v)
        pltpu.sync_copy(cidx_hbm.at[wid], cidx_v)

        def issue_g(j, b):
            pltpu.async_copy(h_hbm.at[ridx_v.at[j]], bufrs[b], sgs[b])
            pltpu.async_copy(h_hbm.at[cidx_v.at[j]], bufcs[b], sgs[b])

        def drain_g(b):
            pltpu.make_async_copy(h_hbm.at[pl.ds(0, _C)], bufrs[b],
                                  sgs[b]).wait()
            pltpu.make_async_copy(h_hbm.at[pl.ds(0, _C)], bufcs[b],
                                  sgs[b]).wait()

        def write(j, b):
            pltpu.async_copy(bufrs[b], outr_hbm.at[pl.ds(base + j * _C, _C)],
                             sws[b])
            pltpu.async_copy(bufcs[b], outc_hbm.at[pl.ds(base + j * _C, _C)],
                             sws[b])
            pltpu.make_async_copy(bufrs[b], outr_hbm.at[pl.ds(0, _C)],
                                  sws[b]).wait()
            pltpu.make_async_copy(bufcs[b], outc_hbm.at[pl.ds(0, _C)],
                                  sws[b]).wait()

        issue_g(0, 0)

        def pair(j2, carry):
            j = j2 * 2

            @pl.when(j + 1 < _NCH)
            def _():
                issue_g(j + 1, 1)

            drain_g(0)
            write(j, 0)

            @pl.when(j + 2 < _NCH)
            def _():
                issue_g(j + 2, 0)

            @pl.when(j + 1 < _NCH)
            def _():
                drain_g(1)
                write(j + 1, 1)

            return carry

        lax.fori_loop(0, (_NCH + 1) // 2, pair, 0)

    return _sc_gather


_EG = 4
_FG = 8
_FPT = _EO // _FG
_SCH = 128
_EGB = 39936
_ACC = _FPT * _NP


@functools.cache
def _sc_scatter_kernel():
    mesh = plsc.VectorSubcoreMesh(core_axis_name="c", subcore_axis_name="s",
                                  num_cores=_NC, num_subcores=_NS)

    @functools.partial(
        pl.kernel,
        out_type=jax.ShapeDtypeStruct((_NW, _ACC), jnp.float32),
        mesh=mesh,
        compiler_params=pltpu.CompilerParams(needs_layout_passes=False),
        scratch_types=[
            pltpu.VMEM((_ACC,), jnp.float32),
            pltpu.VMEM((_SCH,), jnp.int32),
            pltpu.VMEM((_SCH,), jnp.int32),
            pltpu.VMEM((_FPT, _SCH), jnp.float32),
            pltpu.VMEM((_FPT, _SCH), jnp.float32),
            pltpu.SemaphoreType.DMA,
            pltpu.SemaphoreType.DMA,
            pltpu.SemaphoreType.DMA,
            pltpu.SemaphoreType.DMA,
        ],
    )
    def _sc_scatter(et_hbm, cidx_hbm, out_hbm, acc, idxb0, idxb1,
                    ebuf0, ebuf1, si0, si1, se0, se1):
        t = lax.axis_index("c") * _NS + lax.axis_index("s")
        fg = t // _EG
        eg = t % _EG
        base = eg * _EGB
        f0 = fg * _FPT
        nch = jnp.where(eg == _EG - 1, 314, 312)
        sis = (si0, si1)
        ses = (se0, se1)
        idxbs = (idxb0, idxb1)
        ebufs = (ebuf0, ebuf1)

        def zero(v, carry):
            acc[pl.ds(v * 16, 16)] = jnp.zeros((16,), jnp.float32)
            return carry

        lax.fori_loop(0, _ACC // 16, zero, 0)

        def issue(g, b):
            off = base + g * _SCH
            pltpu.async_copy(cidx_hbm.at[pl.ds(off, _SCH)], idxbs[b], sis[b])
            pltpu.async_copy(et_hbm.at[pl.ds(f0, _FPT), pl.ds(off, _SCH)],
                             ebufs[b], ses[b])

        def drain(b):
            pltpu.make_async_copy(cidx_hbm.at[pl.ds(0, _SCH)], idxbs[b],
                                  sis[b]).wait()
            pltpu.make_async_copy(et_hbm.at[pl.ds(0, _FPT), pl.ds(0, _SCH)],
                                  ebufs[b], ses[b]).wait()

        def compute(b):
            def inner(v, c2):
                r_vec = idxbs[b][pl.ds(v * 16, 16)]
                for k in range(_FPT):
                    vals = ebufs[b][k, pl.ds(v * 16, 16)]
                    plsc.addupdate_scatter(acc, [r_vec + k * _NP], vals)
                return c2

            lax.fori_loop(0, _SCH // 16, inner, 0)

        issue(0, 0)

        def pair(g2, carry):
            g = g2 * 2

            @pl.when(g + 1 < nch)
            def _():
                issue(g + 1, 1)

            drain(0)
            compute(0)

            @pl.when(g + 2 < nch)
            def _():
                issue(g + 2, 0)

            @pl.when(g + 1 < nch)
            def _():
                drain(1)
                compute(1)

            return carry

        lax.fori_loop(0, (nch + 1) // 2, pair, 0)
        pltpu.sync_copy(acc, out_hbm.at[t])

    return _sc_scatter


def _edge_body(hr, hc, ea, w1r, w1c, w1e, b1, w2, b2, w3, b3, w4, b4, out, outt):
    f32 = jnp.float32
    bf = jnp.bfloat16
    x = (jnp.dot(hr[...].astype(bf), w1r[...], preferred_element_type=f32)
         + jnp.dot(hc[...].astype(bf), w1c[...], preferred_element_type=f32)
         + jnp.dot(ea[...].astype(bf), w1e[...], preferred_element_type=f32)
         + b1[...])
    x = jnp.maximum(x, 0.0).astype(bf)
    x = jnp.maximum(jnp.dot(x, w2[...], preferred_element_type=f32) + b2[...],
                    0.0).astype(bf)
    x = jnp.maximum(jnp.dot(x, w3[...], preferred_element_type=f32) + b3[...],
                    0.0).astype(bf)
    e = jnp.dot(x, w4[...], preferred_element_type=f32) + b4[...]
    out[...] = e
    outt[...] = e.T


def _full(shape):
    return pl.BlockSpec(shape, lambda i: (0, 0))


def _edge_mlp(hr_g, hc_g, ea, w1r, w1c, w1e, b1, w2, b2, w3, b3, w4, b4):
    grid = (_E + _EB - 1) // _EB
    return pl.pallas_call(
        _edge_body,
        grid=(grid,),
        in_specs=[
            pl.BlockSpec((_EB, _DN), lambda i: (i, 0)),
            pl.BlockSpec((_EB, _DN), lambda i: (i, 0)),
            pl.BlockSpec((_EB, _DE), lambda i: (i, 0)),
            _full(w1r.shape), _full(w1c.shape), _full(w1e.shape), _full(b1.shape),
            _full(w2.shape), _full(b2.shape),
            _full(w3.shape), _full(b3.shape),
            _full(w4.shape), _full(b4.shape),
        ],
        out_specs=[
            pl.BlockSpec((_EB, _EO), lambda i: (i, 0)),
            pl.BlockSpec((_EO, _EB), lambda i: (0, i)),
        ],
        out_shape=[
            jax.ShapeDtypeStruct((_E, _EO), jnp.float32),
            jax.ShapeDtypeStruct((_EO, _E), jnp.float32),
        ],
    )(hr_g, hc_g, ea, w1r, w1c, w1e, b1, w2, b2, w3, b3, w4, b4)


def _node_body(parts, h, w1m, w1h, b1, w2, b2, w3, b3, w4, b4, out):
    f32 = jnp.float32
    p = parts[...].reshape(_FG, _EG, _FPT, _NB)
    msum = jnp.sum(p, axis=1).reshape(_EO, _NB)
    x = (lax.dot_general(msum, w1m[...], (((0,), (0,)), ((), ())),
                         preferred_element_type=f32)
         + jnp.dot(h[...], w1h[...], preferred_element_type=f32)
         + b1[...])
    x = jnp.maximum(x, 0.0)
    x = jnp.maximum(jnp.dot(x, w2[...], preferred_element_type=f32) + b2[...], 0.0)
    x = jnp.maximum(jnp.dot(x, w3[...], preferred_element_type=f32) + b3[...], 0.0)
    out[...] = jnp.dot(x, w4[...], preferred_element_type=f32) + b4[...]


def _node_mlp(parts, h, w1m, w1h, b1, w2, b2, w3, b3, w4, b4):
    grid = _NP // _NB
    return pl.pallas_call(
        _node_body,
        grid=(grid,),
        in_specs=[
            pl.BlockSpec((_NW * _FPT, _NB), lambda i: (0, i)),
            pl.BlockSpec((_NB, _DN), lambda i: (i, 0)),
            _full(w1m.shape), _full(w1h.shape), _full(b1.shape),
            _full(w2.shape), _full(b2.shape),
            _full(w3.shape), _full(b3.shape),
            _full(w4.shape), _full(b4.shape),
        ],
        out_specs=pl.BlockSpec((_NB, _NO), lambda i: (i, 0)),
        out_shape=jax.ShapeDtypeStruct((_N, _NO), jnp.float32),
    )(parts, h, w1m, w1h, b1, w2, b2, w3, b3, w4, b4)


def kernel(h, edge_index, edge_attr, edge_params, node_params):
    row3 = edge_index[0].reshape(_NW, _NCH, _C)
    col3 = edge_index[1].reshape(_NW, _NCH, _C)

    hr_g, hc_g = _sc_gather_kernel()(h, row3, col3)

    (ew1, eb1), (ew2, eb2), (ew3, eb3), (ew4, eb4) = edge_params
    bf = jnp.bfloat16
    e_new, e_newt = _edge_mlp(
        hr_g, hc_g, edge_attr,
        ew1[:_DN].astype(bf), ew1[_DN:2 * _DN].astype(bf),
        ew1[2 * _DN:].astype(bf), eb1.reshape(1, -1),
        ew2.astype(bf), eb2.reshape(1, -1), ew3.astype(bf), eb3.reshape(1, -1),
        ew4.astype(bf), eb4.reshape(1, -1),
    )

    parts = _sc_scatter_kernel()(e_newt, edge_index[1])
    parts = parts.reshape(_NW * _FPT, _NP)

    (nw1, nb1), (nw2, nb2), (nw3, nb3), (nw4, nb4) = node_params
    h_new = _node_mlp(
        parts, h,
        nw1[:_EO], nw1[_EO:], nb1.reshape(1, -1),
        nw2, nb2.reshape(1, -1), nw3, nb3.reshape(1, -1), nw4, nb4.reshape(1, -1),
    )
    return (h_new, e_new)
